# Initial kernel scaffold; baseline (speedup 1.0000x reference)
import jax
import jax.numpy as jnp
from jax.experimental import pallas as pl


def kernel(inputs, embed_table, W, b):
    raise NotImplementedError("write your pallas kernel here")


# speedup vs baseline: 3.1199x; 3.1199x over previous
import functools

import jax
import jax.numpy as jnp
from jax import lax
from jax.experimental import pallas as pl
from jax.experimental.pallas import tpu as pltpu
from jax.experimental.pallas import tpu_sc as plsc

VOCAB = 1000000
D = 32
B = 4096
HIST = 20
C = 1000

_info = plsc.get_sparse_core_info()
_NC, _NS, _L = _info.num_cores, _info.num_subcores, _info.num_lanes
NW = _NC * _NS
B_PER_W = B // NW
IDX_PER_W = B_PER_W * HIST
CHUNK = 128
NCHUNK = IDX_PER_W // CHUNK


def _sc_bag(idx_flat, table):
    mesh = plsc.VectorSubcoreMesh(core_axis_name="c", subcore_axis_name="s")

    @functools.partial(
        pl.kernel,
        mesh=mesh,
        out_type=jax.ShapeDtypeStruct((B, D), jnp.float32),
        scratch_types=[
            pltpu.VMEM((IDX_PER_W,), jnp.int32),
            pltpu.VMEM((IDX_PER_W, D), jnp.float32),
            pltpu.VMEM((B_PER_W, D), jnp.float32),
            pltpu.SemaphoreType.DMA,
        ],
        compiler_params=pltpu.CompilerParams(use_tc_tiling_on_sc=False),
    )
    def k(idx_ref, table_ref, bag_ref, idx_v, rows_v, bag_v, sem):
        wid = lax.axis_index("s") * _NC + lax.axis_index("c")
        base = wid * IDX_PER_W
        pltpu.sync_copy(idx_ref.at[pl.ds(base, IDX_PER_W)], idx_v)
        copies = []
        for c in range(NCHUNK):
            copies.append(pltpu.async_copy(
                table_ref.at[idx_v.at[pl.ds(c * CHUNK, CHUNK)]],
                rows_v.at[pl.ds(c * CHUNK, CHUNK)],
                sem))
        for cp in copies:
            cp.wait()

        def body(i, carry):
            r0 = i * HIST
            acc0 = rows_v[r0, 0:16]
            acc1 = rows_v[r0, 16:32]
            for l in range(1, HIST):
                acc0 = acc0 + rows_v[r0 + l, 0:16]
                acc1 = acc1 + rows_v[r0 + l, 16:32]
            bag_v[i, 0:16] = acc0
            bag_v[i, 16:32] = acc1
            return carry

        lax.fori_loop(0, B_PER_W, body, 0)
        pltpu.sync_copy(bag_v, bag_ref.at[pl.ds(wid * B_PER_W, B_PER_W)])

    return k(idx_flat, table)


def _tc_project(bag, W, b_scaled):
    BM = 512

    def mm(bag_ref, w_ref, b_ref, out_ref):
        acc = lax.dot_general(
            bag_ref[...], w_ref[...],
            (((1,), (1,)), ((), ())),
            preferred_element_type=jnp.float32)
        out_ref[...] = acc + b_ref[...]

    return pl.pallas_call(
        mm,
        grid=(B // BM,),
        in_specs=[
            pl.BlockSpec((BM, D), lambda i: (i, 0)),
            pl.BlockSpec((C, D), lambda i: (0, 0)),
            pl.BlockSpec((1, C), lambda i: (0, 0)),
        ],
        out_specs=pl.BlockSpec((BM, C), lambda i: (i, 0)),
        out_shape=jax.ShapeDtypeStruct((B, C), jnp.float32),
    )(bag, W, b_scaled)


def kernel(inputs, embed_table, W, b):
    idx_flat = inputs.reshape(-1).astype(jnp.int32)
    bag = _sc_bag(idx_flat, embed_table)
    b_scaled = (b * jnp.float32(HIST)).reshape(1, C)
    return _tc_project(bag, W, b_scaled)
